# baseline (speedup 1.0000x reference)
import functools

import jax
import jax.numpy as jnp
from jax import lax
from jax.experimental import pallas as pl
from jax.experimental.pallas import tpu as pltpu
from jax.experimental.pallas import tpu_sc as plsc

NBUF = 4
CHUNK = 4


@functools.lru_cache(maxsize=None)
def _embed_lookup(B: int, H: int, V: int, D: int):
    info = plsc.get_sparse_core_info()
    nw = info.num_cores * info.num_subcores
    rows_per_w = B // nw
    n_idx_w = rows_per_w * H
    stream_len = CHUNK * H
    n_outer = rows_per_w // (NBUF * CHUNK)
    assert rows_per_w * nw == B
    assert n_outer * NBUF * CHUNK == rows_per_w

    mesh = plsc.VectorSubcoreMesh(core_axis_name="c", subcore_axis_name="s")

    @functools.partial(
        pl.kernel,
        mesh=mesh,
        out_type=jax.ShapeDtypeStruct((B * H, D), jnp.float32),
        scratch_types=[
            pltpu.VMEM((n_idx_w,), jnp.int32),
            pltpu.VMEM((NBUF, stream_len, D), jnp.float32),
        ]
        + [pltpu.SemaphoreType.DMA] * (2 * NBUF),
        compiler_params=pltpu.CompilerParams(use_tc_tiling_on_sc=False),
    )
    def k(idx_hbm, table_hbm, out_hbm, idx_v, rows_v, *sems):
        gsems, wsems = sems[:NBUF], sems[NBUF:]
        wid = lax.axis_index("s") * info.num_cores + lax.axis_index("c")
        base = wid * n_idx_w

        pltpu.sync_copy(idx_hbm.at[pl.ds(base, n_idx_w)], idx_v)

        def fire_gather(c, b):
            pltpu.async_copy(
                table_hbm.at[idx_v.at[pl.ds(c * stream_len, stream_len)]],
                rows_v.at[b],
                gsems[b],
            )

        def wait_gather(b):
            pltpu.make_async_copy(
                table_hbm.at[idx_v.at[pl.ds(0, stream_len)]],
                rows_v.at[b],
                gsems[b],
            ).wait()

        def fire_write(c, b):
            pltpu.async_copy(
                rows_v.at[b],
                out_hbm.at[pl.ds(base + c * stream_len, stream_len)],
                wsems[b],
            )

        def wait_write(b):
            pltpu.make_async_copy(
                rows_v.at[b],
                out_hbm.at[pl.ds(base, stream_len)],
                wsems[b],
            ).wait()

        for b in range(NBUF):
            fire_gather(b, b)
        for b in range(NBUF):
            wait_gather(b)
            fire_write(b, b)

        def body(i, carry):
            for b in range(NBUF):
                wait_write(b)
                fire_gather(NBUF * i + b, b)
            for b in range(NBUF):
                wait_gather(b)
                fire_write(NBUF * i + b, b)
            return carry

        lax.fori_loop(1, n_outer, body, 0, unroll=False)

        for b in range(NBUF):
            wait_write(b)

    return k


def kernel(inputs, table):
    b, h = inputs.shape
    v, d = table.shape
    flat = _embed_lookup(b, h, v, d)(
        inputs.reshape(b * h).astype(jnp.int32), table
    )
    return flat.reshape(b, h, d)


# speedup vs baseline: 1.0715x; 1.0715x over previous
import functools

import jax
import jax.numpy as jnp
from jax import lax
from jax.experimental import pallas as pl
from jax.experimental.pallas import tpu as pltpu
from jax.experimental.pallas import tpu_sc as plsc

NBUF = 4
CHUNK = 4
XROWS = 512


@functools.lru_cache(maxsize=None)
def _detile(V: int, D: int):
    lanes_per_row = 128 // D
    sup = XROWS * lanes_per_row
    n_sup = (V + sup - 1) // sup

    def body(t_ref, o_ref):
        t = t_ref[...]
        for k in range(lanes_per_row):
            o_ref[:, k * D:(k + 1) * D] = t[:, k * XROWS:(k + 1) * XROWS].T

    return pl.pallas_call(
        body,
        grid=(n_sup,),
        in_specs=[pl.BlockSpec((D, sup), lambda j: (0, j))],
        out_specs=pl.BlockSpec((XROWS, 128), lambda j: (j, 0)),
        out_shape=jax.ShapeDtypeStruct((n_sup * XROWS, 128), jnp.float32),
    )


@functools.lru_cache(maxsize=None)
def _remap_idx(B: int, H: int):
    sup = XROWS * (128 // 32)

    def body(i_ref, o_ref):
        r = i_ref[...]
        o_ref[...] = (
            (r & ~(sup - 1))
            | ((r & (XROWS - 1)) << 2)
            | ((r >> 9) & (sup // XROWS - 1))
        )

    return pl.pallas_call(
        body,
        out_shape=jax.ShapeDtypeStruct((B, H), jnp.int32),
    )


@functools.lru_cache(maxsize=None)
def _embed_lookup(B: int, H: int, V: int, D: int):
    info = plsc.get_sparse_core_info()
    nw = info.num_cores * info.num_subcores
    rows_per_w = B // nw
    n_idx_w = rows_per_w * H
    stream_len = CHUNK * H
    n_outer = rows_per_w // (NBUF * CHUNK)
    assert rows_per_w * nw == B
    assert n_outer * NBUF * CHUNK == rows_per_w

    mesh = plsc.VectorSubcoreMesh(core_axis_name="c", subcore_axis_name="s")

    @functools.partial(
        pl.kernel,
        mesh=mesh,
        out_type=jax.ShapeDtypeStruct((B * H, D), jnp.float32),
        scratch_types=[
            pltpu.VMEM((n_idx_w,), jnp.int32),
            pltpu.VMEM((NBUF, stream_len, D), jnp.float32),
        ]
        + [pltpu.SemaphoreType.DMA] * (2 * NBUF),
        compiler_params=pltpu.CompilerParams(use_tc_tiling_on_sc=False),
    )
    def k(idx_hbm, table_hbm, out_hbm, idx_v, rows_v, *sems):
        gsems, wsems = sems[:NBUF], sems[NBUF:]
        wid = lax.axis_index("s") * info.num_cores + lax.axis_index("c")
        base = wid * n_idx_w

        pltpu.sync_copy(idx_hbm.at[pl.ds(base, n_idx_w)], idx_v)

        def fire_gather(c, b):
            pltpu.async_copy(
                table_hbm.at[idx_v.at[pl.ds(c * stream_len, stream_len)]],
                rows_v.at[b],
                gsems[b],
            )

        def wait_gather(b):
            pltpu.make_async_copy(
                table_hbm.at[idx_v.at[pl.ds(0, stream_len)]],
                rows_v.at[b],
                gsems[b],
            ).wait()

        def fire_write(c, b):
            pltpu.async_copy(
                rows_v.at[b],
                out_hbm.at[pl.ds(base + c * stream_len, stream_len)],
                wsems[b],
            )

        def wait_write(b):
            pltpu.make_async_copy(
                rows_v.at[b],
                out_hbm.at[pl.ds(base, stream_len)],
                wsems[b],
            ).wait()

        for b in range(NBUF):
            fire_gather(b, b)
        for b in range(NBUF):
            wait_gather(b)
            fire_write(b, b)

        def body(i, carry):
            for b in range(NBUF):
                wait_write(b)
                fire_gather(NBUF * i + b, b)
            for b in range(NBUF):
                wait_gather(b)
                fire_write(NBUF * i + b, b)
            return carry

        lax.fori_loop(1, n_outer, body, 0, unroll=False)

        for b in range(NBUF):
            wait_write(b)

    return k


def kernel(inputs, table):
    b, h = inputs.shape
    v, d = table.shape
    t_perm = _detile(v, d)(table.T)
    vrows = t_perm.shape[0] * (128 // d)
    idx_v = _remap_idx(b, h)(inputs.astype(jnp.int32))
    flat = _embed_lookup(b, h, vrows, d)(
        idx_v.reshape(b * h),
        t_perm.reshape(vrows, d),
    )
    return flat.reshape(b, h, d)
